# baseline (speedup 1.0000x reference)
import functools

import jax
import jax.numpy as jnp
from jax.experimental import pallas as pl
from jax.experimental.pallas import tpu as pltpu

_B = 16384
_D = 64
_BB = 256
_NB = _B // _BB
_NSEM = 8


def _tc_body(uidx, iidx, ut, it, out_ref, ub, ib, sems):
    g = pl.program_id(0)
    for j in range(_BB):
        u = uidx[g * _BB + j]
        i = iidx[g * _BB + j]
        pltpu.make_async_copy(
            ut.at[pl.ds(u, 1), :], ub.at[pl.ds(j, 1), :], sems.at[j % _NSEM]
        ).start()
        pltpu.make_async_copy(
            it.at[pl.ds(i, 1), :], ib.at[pl.ds(j, 1), :], sems.at[_NSEM + j % _NSEM]
        ).start()
    for j in range(_BB):
        pltpu.make_async_copy(
            ut.at[pl.ds(0, 1), :], ub.at[pl.ds(j, 1), :], sems.at[j % _NSEM]
        ).wait()
        pltpu.make_async_copy(
            it.at[pl.ds(0, 1), :], ib.at[pl.ds(j, 1), :], sems.at[_NSEM + j % _NSEM]
        ).wait()
    out_ref[...] = jnp.sum(ub[...] * ib[...], axis=1)


_grid_spec = pltpu.PrefetchScalarGridSpec(
    num_scalar_prefetch=2,
    grid=(_NB,),
    in_specs=[
        pl.BlockSpec(memory_space=pltpu.HBM),
        pl.BlockSpec(memory_space=pltpu.HBM),
    ],
    out_specs=pl.BlockSpec((_BB,), lambda g, uidx, iidx: (g,)),
    scratch_shapes=[
        pltpu.VMEM((_BB, _D), jnp.float32),
        pltpu.VMEM((_BB, _D), jnp.float32),
        pltpu.SemaphoreType.DMA((2 * _NSEM,)),
    ],
)

_tc_gather = pl.pallas_call(
    _tc_body,
    grid_spec=_grid_spec,
    out_shape=jax.ShapeDtypeStruct((_B,), jnp.float32),
)


def kernel(user, item, user_table, item_table):
    return _tc_gather(user, item, user_table, item_table)
